# baseline (speedup 1.0000x reference)
import dataclasses
import functools

import jax
import jax.numpy as jnp
from jax import lax
from jax.experimental import pallas as pl
from jax.experimental.pallas import tpu as pltpu
from jax.experimental.pallas import tpu_sc as plsc

POOL = 7
NC = 2
NS = 16
NW = NC * NS
LANES = 16
CH = 32


def _build_sc_kernel(n_rows, depth, s_out, s_pad, chunks_per_tile):
    per_tile = s_pad // NW
    d2 = 2 * depth
    last_cb = s_out - CH

    cp = pltpu.CompilerParams()
    if "needs_layout_passes" in pltpu.CompilerParams.__dataclass_fields__:
        cp = dataclasses.replace(cp, needs_layout_passes=False)

    @functools.partial(
        pl.kernel,
        mesh=plsc.VectorSubcoreMesh(core_axis_name="c", subcore_axis_name="s"),
        compiler_params=cp,
        out_type=jax.ShapeDtypeStruct((s_out, depth), jnp.float32),
        scratch_types=[
            pltpu.VMEM((per_tile,), jnp.int32),
            pltpu.VMEM((per_tile,), jnp.int32),
            pltpu.VMEM((per_tile,), jnp.float32),
            pltpu.VMEM((per_tile,), jnp.float32),
            pltpu.VMEM((per_tile,), jnp.float32),
            pltpu.VMEM((per_tile,), jnp.float32),
            pltpu.VMEM((2, CH, d2), jnp.float32),
            pltpu.VMEM((2, CH, d2), jnp.float32),
            pltpu.VMEM((2, CH, depth), jnp.float32),
            pltpu.SemaphoreType.DMA,
            pltpu.SemaphoreType.DMA,
        ],
    )
    def sc_kernel(table_hbm, idxt_hbm, idxb_hbm, w00_hbm, w01_hbm, w10_hbm,
                  w11_hbm, out_hbm, idxt_v, idxb_v, w00_v, w01_v, w10_v, w11_v,
                  top_v, bot_v, out_v, gsem, osem):
        wid = lax.axis_index("s") * NC + lax.axis_index("c")
        base = wid * per_tile
        pltpu.sync_copy(idxt_hbm.at[wid], idxt_v)
        pltpu.sync_copy(idxb_hbm.at[wid], idxb_v)
        pltpu.sync_copy(w00_hbm.at[wid], w00_v)
        pltpu.sync_copy(w01_hbm.at[wid], w01_v)
        pltpu.sync_copy(w10_hbm.at[wid], w10_v)
        pltpu.sync_copy(w11_hbm.at[wid], w11_v)

        def issue_gathers(g, b):
            sl = pl.ds(g * CH, CH)
            pltpu.async_copy(table_hbm.at[idxt_v.at[sl]], top_v.at[b], gsem)
            pltpu.async_copy(table_hbm.at[idxb_v.at[sl]], bot_v.at[b], gsem)

        def wait_gathers(b):
            dummy = table_hbm.at[pl.ds(0, CH)]
            pltpu.make_async_copy(dummy, top_v.at[b], gsem).wait()
            pltpu.make_async_copy(dummy, bot_v.at[b], gsem).wait()

        def wait_store(b):
            pltpu.make_async_copy(
                out_v.at[b], out_hbm.at[pl.ds(0, CH)], osem).wait()

        issue_gathers(0, 0)

        @pl.loop(0, chunks_per_tile)
        def _chunk(g):
            b = lax.rem(g, 2)
            cb = jnp.minimum(base + g * CH, last_cb)

            @pl.when(g + 1 < chunks_per_tile)
            def _():
                issue_gathers(g + 1, 1 - b)

            wait_gathers(b)

            @pl.when(g >= 2)
            def _():
                wait_store(b)

            @pl.loop(0, CH)
            def _sample(i):
                sv = jnp.full((LANES,), g * CH + i, jnp.int32)
                w00 = plsc.load_gather(w00_v, [sv])
                w01 = plsc.load_gather(w01_v, [sv])
                w10 = plsc.load_gather(w10_v, [sv])
                w11 = plsc.load_gather(w11_v, [sv])

                for c in range(0, depth, LANES):
                    tl = top_v[b, i, pl.ds(c, LANES)]
                    tr = top_v[b, i, pl.ds(c + depth, LANES)]
                    bl = bot_v[b, i, pl.ds(c, LANES)]
                    br = bot_v[b, i, pl.ds(c + depth, LANES)]
                    out_v[b, i, pl.ds(c, LANES)] = (
                        w00 * tl + w01 * tr + w10 * bl + w11 * br)

            pltpu.async_copy(out_v.at[b], out_hbm.at[pl.ds(cb, CH)], osem)

        wait_store(0)
        wait_store(1)

    return sc_kernel


def kernel(inputs, proposals):
    B, H, W, C = inputs.shape
    NB = proposals.shape[1]
    n = B * NB

    boxes = proposals.reshape(-1, 4)
    y1, x1, y2, x2 = boxes[:, 0], boxes[:, 1], boxes[:, 2], boxes[:, 3]
    grid = jnp.arange(POOL, dtype=jnp.float32)
    h_scale = (y2 - y1) * (H - 1) / (POOL - 1)
    w_scale = (x2 - x1) * (W - 1) / (POOL - 1)
    in_y = y1[:, None] * (H - 1) + grid[None, :] * h_scale[:, None]
    in_x = x1[:, None] * (W - 1) + grid[None, :] * w_scale[:, None]
    valid_y = (in_y >= 0) & (in_y <= H - 1)
    valid_x = (in_x >= 0) & (in_x <= W - 1)
    top_y = jnp.floor(in_y)
    left_x = jnp.floor(in_x)
    yl = in_y - top_y
    xl = in_x - left_x
    ti = jnp.clip(top_y, 0, H - 1).astype(jnp.int32)
    bi = jnp.clip(jnp.ceil(in_y), 0, H - 1).astype(jnp.int32)
    li = jnp.clip(left_x, 0, W - 1).astype(jnp.int32)

    bflat = (jnp.arange(n, dtype=jnp.int32) // NB) * (H * W)
    idx_t = bflat[:, None, None] + ti[:, :, None] * W + li[:, None, :]
    idx_b = bflat[:, None, None] + bi[:, :, None] * W + li[:, None, :]

    v = (valid_y[:, :, None] & valid_x[:, None, :]).astype(jnp.float32)
    ylc = yl[:, :, None]
    xlc = xl[:, None, :]
    w00 = (1.0 - ylc) * (1.0 - xlc) * v
    w01 = (1.0 - ylc) * xlc * v
    w10 = ylc * (1.0 - xlc) * v
    w11 = ylc * xlc * v

    s = n * POOL * POOL
    s_pad = ((s + NW * CH - 1) // (NW * CH)) * (NW * CH)
    chunks_per_tile = s_pad // NW // CH
    per_tile = s_pad // NW

    cb = jnp.minimum(jnp.arange(s_pad // CH) * CH, s - CH)
    src = (cb[:, None] + jnp.arange(CH)[None, :]).reshape(-1)

    def tiled(a):
        return a.reshape(-1)[src].reshape(NW, per_tile)

    flat = inputs.reshape(B * H * W, C)
    table = jnp.concatenate([flat, jnp.roll(flat, -1, axis=0)], axis=1)

    sc_kernel = _build_sc_kernel(B * H * W, C, s, s_pad, chunks_per_tile)
    out = sc_kernel(table, tiled(idx_t), tiled(idx_b),
                    tiled(w00), tiled(w01), tiled(w10), tiled(w11))
    return out.reshape(B, NB, POOL, POOL, C)


# speedup vs baseline: 1.0639x; 1.0639x over previous
import dataclasses
import functools

import jax
import jax.numpy as jnp
from jax import lax
from jax.experimental import pallas as pl
from jax.experimental.pallas import tpu as pltpu
from jax.experimental.pallas import tpu_sc as plsc

POOL = 7
NC = 2
NS = 16
NW = NC * NS
LANES = 16
CH = 32


def _build_sc_kernel(n_rows, depth, s_out, s_pad, chunks_per_tile):
    per_tile = s_pad // NW
    d2 = 2 * depth
    last_cb = s_out - CH

    cp = pltpu.CompilerParams()
    if "needs_layout_passes" in pltpu.CompilerParams.__dataclass_fields__:
        cp = dataclasses.replace(cp, needs_layout_passes=False)

    @functools.partial(
        pl.kernel,
        mesh=plsc.VectorSubcoreMesh(core_axis_name="c", subcore_axis_name="s"),
        compiler_params=cp,
        out_type=jax.ShapeDtypeStruct((s_out, depth), jnp.float32),
        scratch_types=[
            pltpu.VMEM((per_tile,), jnp.int32),
            pltpu.VMEM((per_tile,), jnp.int32),
            pltpu.VMEM((per_tile,), jnp.float32),
            pltpu.VMEM((per_tile,), jnp.float32),
            pltpu.VMEM((per_tile,), jnp.float32),
            pltpu.VMEM((per_tile,), jnp.float32),
            pltpu.VMEM((2, CH, d2), jnp.float32),
            pltpu.VMEM((2, CH, d2), jnp.float32),
            pltpu.VMEM((2, CH, depth), jnp.float32),
            pltpu.SemaphoreType.DMA,
            pltpu.SemaphoreType.DMA,
        ],
    )
    def sc_kernel(table_hbm, idxt_hbm, idxb_hbm, w00_hbm, w01_hbm, w10_hbm,
                  w11_hbm, out_hbm, idxt_v, idxb_v, w00_v, w01_v, w10_v, w11_v,
                  top_v, bot_v, out_v, gsem, osem):
        wid = lax.axis_index("s") * NC + lax.axis_index("c")
        base = wid * per_tile
        pltpu.sync_copy(idxt_hbm.at[wid], idxt_v)
        pltpu.sync_copy(idxb_hbm.at[wid], idxb_v)
        pltpu.sync_copy(w00_hbm.at[wid], w00_v)
        pltpu.sync_copy(w01_hbm.at[wid], w01_v)
        pltpu.sync_copy(w10_hbm.at[wid], w10_v)
        pltpu.sync_copy(w11_hbm.at[wid], w11_v)

        def issue_gathers(g, b):
            sl = pl.ds(g * CH, CH)
            pltpu.async_copy(table_hbm.at[idxt_v.at[sl]], top_v.at[b], gsem)
            pltpu.async_copy(table_hbm.at[idxb_v.at[sl]], bot_v.at[b], gsem)

        def wait_gathers(b):
            dummy = table_hbm.at[pl.ds(0, CH)]
            pltpu.make_async_copy(dummy, top_v.at[b], gsem).wait()
            pltpu.make_async_copy(dummy, bot_v.at[b], gsem).wait()

        def wait_store(b):
            pltpu.make_async_copy(
                out_v.at[b], out_hbm.at[pl.ds(0, CH)], osem).wait()

        issue_gathers(0, 0)

        @pl.loop(0, chunks_per_tile)
        def _chunk(g):
            b = lax.rem(g, 2)
            cb = jnp.minimum(base + g * CH, last_cb)

            @pl.when(g + 1 < chunks_per_tile)
            def _():
                issue_gathers(g + 1, 1 - b)

            wait_gathers(b)

            @pl.when(g >= 2)
            def _():
                wait_store(b)

            @pl.loop(0, CH)
            def _sample(i):
                sv = jnp.full((LANES,), g * CH + i, jnp.int32)
                w00 = plsc.load_gather(w00_v, [sv])
                w01 = plsc.load_gather(w01_v, [sv])
                w10 = plsc.load_gather(w10_v, [sv])
                w11 = plsc.load_gather(w11_v, [sv])

                for c in range(0, depth, LANES):
                    tl = top_v[b, i, pl.ds(c, LANES)]
                    tr = top_v[b, i, pl.ds(c + depth, LANES)]
                    bl = bot_v[b, i, pl.ds(c, LANES)]
                    br = bot_v[b, i, pl.ds(c + depth, LANES)]
                    out_v[b, i, pl.ds(c, LANES)] = (
                        w00 * tl + w01 * tr + w10 * bl + w11 * br)

            pltpu.async_copy(out_v.at[b], out_hbm.at[pl.ds(cb, CH)], osem)

        wait_store(0)
        wait_store(1)

    return sc_kernel


def kernel(inputs, proposals):
    B, H, W, C = inputs.shape
    NB = proposals.shape[1]
    n = B * NB

    boxes = proposals.reshape(-1, 4)
    y1, x1, y2, x2 = boxes[:, 0], boxes[:, 1], boxes[:, 2], boxes[:, 3]
    grid = jnp.arange(POOL, dtype=jnp.float32)
    h_scale = (y2 - y1) * (H - 1) / (POOL - 1)
    w_scale = (x2 - x1) * (W - 1) / (POOL - 1)
    in_y = y1[:, None] * (H - 1) + grid[None, :] * h_scale[:, None]
    in_x = x1[:, None] * (W - 1) + grid[None, :] * w_scale[:, None]
    valid_y = (in_y >= 0) & (in_y <= H - 1)
    valid_x = (in_x >= 0) & (in_x <= W - 1)
    top_y = jnp.floor(in_y)
    left_x = jnp.floor(in_x)
    yl = in_y - top_y
    xl = in_x - left_x
    ti = jnp.clip(top_y, 0, H - 1).astype(jnp.int32)
    bi = jnp.clip(jnp.ceil(in_y), 0, H - 1).astype(jnp.int32)
    li = jnp.clip(left_x, 0, W - 1).astype(jnp.int32)

    bflat = (jnp.arange(n, dtype=jnp.int32) // NB) * (H * W)
    idx_t = bflat[:, None, None] + ti[:, :, None] * W + li[:, None, :]
    idx_b = bflat[:, None, None] + bi[:, :, None] * W + li[:, None, :]

    v = (valid_y[:, :, None] & valid_x[:, None, :]).astype(jnp.float32)
    ylc = yl[:, :, None]
    xlc = xl[:, None, :]
    w00 = (1.0 - ylc) * (1.0 - xlc) * v
    w01 = (1.0 - ylc) * xlc * v
    w10 = ylc * (1.0 - xlc) * v
    w11 = ylc * xlc * v

    s = n * POOL * POOL
    s_pad = ((s + NW * CH - 1) // (NW * CH)) * (NW * CH)
    chunks_per_tile = s_pad // NW // CH
    per_tile = s_pad // NW

    n_id = (s - CH) // CH + 1
    rep = s_pad // CH - n_id

    def tiled(a):
        a = a.reshape(-1)
        a = jnp.concatenate([a[:n_id * CH], jnp.tile(a[s - CH:s], rep)])
        return a.reshape(NW, per_tile)

    flat = inputs.reshape(B * H * W, C)
    table = jnp.concatenate([flat, jnp.roll(flat, -1, axis=0)], axis=1)

    sc_kernel = _build_sc_kernel(B * H * W, C, s, s_pad, chunks_per_tile)
    out = sc_kernel(table, tiled(idx_t), tiled(idx_b),
                    tiled(w00), tiled(w01), tiled(w10), tiled(w11))
    return out.reshape(B, NB, POOL, POOL, C)


# speedup vs baseline: 1.3339x; 1.2538x over previous
import dataclasses
import functools

import jax
import jax.numpy as jnp
from jax import lax
from jax.experimental import pallas as pl
from jax.experimental.pallas import tpu as pltpu
from jax.experimental.pallas import tpu_sc as plsc

POOL = 7
NC = 2
NS = 16
NW = NC * NS
LANES = 16
CH = 32


def _build_sc_kernel(n_rows, depth, s_out, s_pad, chunks_per_tile):
    per_tile = s_pad // NW
    d2 = 2 * depth
    last_cb = s_out - CH

    cp = pltpu.CompilerParams()
    if "needs_layout_passes" in pltpu.CompilerParams.__dataclass_fields__:
        cp = dataclasses.replace(cp, needs_layout_passes=False)

    @functools.partial(
        pl.kernel,
        mesh=plsc.VectorSubcoreMesh(core_axis_name="c", subcore_axis_name="s"),
        compiler_params=cp,
        out_type=jax.ShapeDtypeStruct((s_out, depth), jnp.float32),
        scratch_types=[
            pltpu.VMEM((per_tile,), jnp.int32),
            pltpu.VMEM((per_tile,), jnp.int32),
            pltpu.VMEM((per_tile,), jnp.float32),
            pltpu.VMEM((per_tile,), jnp.float32),
            pltpu.VMEM((per_tile,), jnp.float32),
            pltpu.VMEM((per_tile,), jnp.float32),
            pltpu.VMEM((2, CH, d2), jnp.float32),
            pltpu.VMEM((2, CH, d2), jnp.float32),
            pltpu.VMEM((2, CH, depth), jnp.float32),
            pltpu.SemaphoreType.DMA,
            pltpu.SemaphoreType.DMA,
        ],
    )
    def sc_kernel(table_hbm, idxt_hbm, idxb_hbm, w00_hbm, w01_hbm, w10_hbm,
                  w11_hbm, out_hbm, idxt_v, idxb_v, w00_v, w01_v, w10_v, w11_v,
                  top_v, bot_v, out_v, gsem, osem):
        wid = lax.axis_index("s") * NC + lax.axis_index("c")
        base = wid * per_tile
        pltpu.sync_copy(idxt_hbm.at[wid], idxt_v)
        pltpu.sync_copy(idxb_hbm.at[wid], idxb_v)
        pltpu.sync_copy(w00_hbm.at[wid], w00_v)
        pltpu.sync_copy(w01_hbm.at[wid], w01_v)
        pltpu.sync_copy(w10_hbm.at[wid], w10_v)
        pltpu.sync_copy(w11_hbm.at[wid], w11_v)

        def issue_gathers(g, b):
            sl = pl.ds(g * CH, CH)
            pltpu.async_copy(table_hbm.at[idxt_v.at[sl]], top_v.at[b], gsem)
            pltpu.async_copy(table_hbm.at[idxb_v.at[sl]], bot_v.at[b], gsem)

        def wait_gathers(b):
            dummy = table_hbm.at[pl.ds(0, CH)]
            pltpu.make_async_copy(dummy, top_v.at[b], gsem).wait()
            pltpu.make_async_copy(dummy, bot_v.at[b], gsem).wait()

        def wait_store(b):
            pltpu.make_async_copy(
                out_v.at[b], out_hbm.at[pl.ds(0, CH)], osem).wait()

        issue_gathers(0, 0)

        @pl.loop(0, chunks_per_tile)
        def _chunk(g):
            b = lax.rem(g, 2)
            cb = jnp.minimum(base + g * CH, last_cb)

            @pl.when(g + 1 < chunks_per_tile)
            def _():
                issue_gathers(g + 1, 1 - b)

            wait_gathers(b)

            @pl.when(g >= 2)
            def _():
                wait_store(b)

            @pl.loop(0, CH)
            def _sample(i):
                sv = jnp.full((LANES,), g * CH + i, jnp.int32)
                w00 = plsc.load_gather(w00_v, [sv])
                w01 = plsc.load_gather(w01_v, [sv])
                w10 = plsc.load_gather(w10_v, [sv])
                w11 = plsc.load_gather(w11_v, [sv])

                for c in range(0, depth, LANES):
                    tl = top_v[b, i, pl.ds(c, LANES)]
                    tr = top_v[b, i, pl.ds(c + depth, LANES)]
                    bl = bot_v[b, i, pl.ds(c, LANES)]
                    br = bot_v[b, i, pl.ds(c + depth, LANES)]
                    out_v[b, i, pl.ds(c, LANES)] = (
                        w00 * tl + w01 * tr + w10 * bl + w11 * br)

            pltpu.async_copy(out_v.at[b], out_hbm.at[pl.ds(cb, CH)], osem)

        wait_store(0)
        wait_store(1)

    return sc_kernel


def _tc_relayout(flat_out, B, NB, C):
    BB = 8
    nblk = NB // BB

    def body(x_ref, o_ref):
        o_ref[...] = x_ref[...].reshape(o_ref.shape)

    return pl.pallas_call(
        body,
        grid=(B * nblk,),
        in_specs=[pl.BlockSpec((BB * POOL * POOL, C), lambda i: (i, 0))],
        out_specs=pl.BlockSpec((1, BB, POOL, POOL, C),
                               lambda i: (i // nblk, i % nblk, 0, 0, 0)),
        out_shape=jax.ShapeDtypeStruct((B, NB, POOL, POOL, C), jnp.float32),
    )(flat_out)


def kernel(inputs, proposals):
    B, H, W, C = inputs.shape
    NB = proposals.shape[1]
    n = B * NB

    boxes = proposals.reshape(-1, 4)
    y1, x1, y2, x2 = boxes[:, 0], boxes[:, 1], boxes[:, 2], boxes[:, 3]
    grid = jnp.arange(POOL, dtype=jnp.float32)
    h_scale = (y2 - y1) * (H - 1) / (POOL - 1)
    w_scale = (x2 - x1) * (W - 1) / (POOL - 1)
    in_y = y1[:, None] * (H - 1) + grid[None, :] * h_scale[:, None]
    in_x = x1[:, None] * (W - 1) + grid[None, :] * w_scale[:, None]
    valid_y = (in_y >= 0) & (in_y <= H - 1)
    valid_x = (in_x >= 0) & (in_x <= W - 1)
    top_y = jnp.floor(in_y)
    left_x = jnp.floor(in_x)
    yl = in_y - top_y
    xl = in_x - left_x
    ti = jnp.clip(top_y, 0, H - 1).astype(jnp.int32)
    bi = jnp.clip(jnp.ceil(in_y), 0, H - 1).astype(jnp.int32)
    li = jnp.clip(left_x, 0, W - 1).astype(jnp.int32)

    bflat = (jnp.arange(n, dtype=jnp.int32) // NB) * (H * W)
    idx_t = bflat[:, None, None] + ti[:, :, None] * W + li[:, None, :]
    idx_b = bflat[:, None, None] + bi[:, :, None] * W + li[:, None, :]

    v = (valid_y[:, :, None] & valid_x[:, None, :]).astype(jnp.float32)
    ylc = yl[:, :, None]
    xlc = xl[:, None, :]
    w00 = (1.0 - ylc) * (1.0 - xlc) * v
    w01 = (1.0 - ylc) * xlc * v
    w10 = ylc * (1.0 - xlc) * v
    w11 = ylc * xlc * v

    s = n * POOL * POOL
    s_pad = ((s + NW * CH - 1) // (NW * CH)) * (NW * CH)
    chunks_per_tile = s_pad // NW // CH
    per_tile = s_pad // NW

    n_id = (s - CH) // CH + 1
    rep = s_pad // CH - n_id

    def tiled(a):
        a = a.reshape(-1)
        a = jnp.concatenate([a[:n_id * CH], jnp.tile(a[s - CH:s], rep)])
        return a.reshape(NW, per_tile)

    flat = inputs.reshape(B * H * W, C)
    table = jnp.concatenate([flat, jnp.roll(flat, -1, axis=0)], axis=1)

    sc_kernel = _build_sc_kernel(B * H * W, C, s, s_pad, chunks_per_tile)
    out = sc_kernel(table, tiled(idx_t), tiled(idx_b),
                    tiled(w00), tiled(w01), tiled(w10), tiled(w11))
    return _tc_relayout(out, B, NB, C)


# speedup vs baseline: 1.8995x; 1.4239x over previous
import dataclasses
import functools

import jax
import jax.numpy as jnp
from jax import lax
from jax.experimental import pallas as pl
from jax.experimental.pallas import tpu as pltpu
from jax.experimental.pallas import tpu_sc as plsc

POOL = 7
NC = 2
NS = 16
NW = NC * NS
LANES = 16
CH = 32


def _build_sc_kernel(n_rows, depth, s_out, s_pad, chunks_per_tile):
    per_tile = s_pad // NW
    d2 = 2 * depth
    last_cb = s_out - CH

    cp = pltpu.CompilerParams()
    if "needs_layout_passes" in pltpu.CompilerParams.__dataclass_fields__:
        cp = dataclasses.replace(cp, needs_layout_passes=False)

    @functools.partial(
        pl.kernel,
        mesh=plsc.VectorSubcoreMesh(core_axis_name="c", subcore_axis_name="s"),
        compiler_params=cp,
        out_type=jax.ShapeDtypeStruct((s_out, depth), jnp.float32),
        scratch_types=[
            pltpu.VMEM((per_tile,), jnp.int32),
            pltpu.VMEM((per_tile,), jnp.int32),
            pltpu.VMEM((per_tile,), jnp.float32),
            pltpu.VMEM((per_tile,), jnp.float32),
            pltpu.VMEM((per_tile,), jnp.float32),
            pltpu.VMEM((per_tile,), jnp.float32),
            pltpu.VMEM((2, CH, depth), jnp.int32),
            pltpu.VMEM((2, CH, depth), jnp.int32),
            pltpu.VMEM((2, CH, depth), jnp.float32),
            pltpu.SemaphoreType.DMA,
            pltpu.SemaphoreType.DMA,
        ],
    )
    def sc_kernel(table_hbm, idxt_hbm, idxb_hbm, w00_hbm, w01_hbm, w10_hbm,
                  w11_hbm, out_hbm, idxt_v, idxb_v, w00_v, w01_v, w10_v, w11_v,
                  top_v, bot_v, out_v, gsem, osem):
        wid = lax.axis_index("s") * NC + lax.axis_index("c")
        base = wid * per_tile
        pltpu.sync_copy(idxt_hbm.at[wid], idxt_v)
        pltpu.sync_copy(idxb_hbm.at[wid], idxb_v)
        pltpu.sync_copy(w00_hbm.at[wid], w00_v)
        pltpu.sync_copy(w01_hbm.at[wid], w01_v)
        pltpu.sync_copy(w10_hbm.at[wid], w10_v)
        pltpu.sync_copy(w11_hbm.at[wid], w11_v)

        def issue_gathers(g, b):
            sl = pl.ds(g * CH, CH)
            pltpu.async_copy(table_hbm.at[idxt_v.at[sl]], top_v.at[b], gsem)
            pltpu.async_copy(table_hbm.at[idxb_v.at[sl]], bot_v.at[b], gsem)

        def wait_gathers(b):
            dummy = table_hbm.at[pl.ds(0, CH)]
            pltpu.make_async_copy(dummy, top_v.at[b], gsem).wait()
            pltpu.make_async_copy(dummy, bot_v.at[b], gsem).wait()

        def wait_store(b):
            pltpu.make_async_copy(
                out_v.at[b], out_hbm.at[pl.ds(0, CH)], osem).wait()

        issue_gathers(0, 0)

        @pl.loop(0, chunks_per_tile)
        def _chunk(g):
            b = lax.rem(g, 2)
            cb = jnp.minimum(base + g * CH, last_cb)

            @pl.when(g + 1 < chunks_per_tile)
            def _():
                issue_gathers(g + 1, 1 - b)

            wait_gathers(b)

            @pl.when(g >= 2)
            def _():
                wait_store(b)

            @pl.loop(0, CH)
            def _sample(i):
                sv = jnp.full((LANES,), g * CH + i, jnp.int32)
                w00 = plsc.load_gather(w00_v, [sv])
                w01 = plsc.load_gather(w01_v, [sv])
                w10 = plsc.load_gather(w10_v, [sv])
                w11 = plsc.load_gather(w11_v, [sv])

                for q in range(depth // (2 * LANES)):
                    c = q * 2 * LANES

                    def pair(ref):
                        v = plsc.bitcast(ref, jnp.bfloat16)
                        return plsc.unpack(
                            v, format=plsc.PackFormat.INTERLEAVED)

                    tlv = pair(top_v[b, i, pl.ds(q * LANES, LANES)])
                    trv = pair(top_v[b, i, pl.ds(depth // 2 + q * LANES, LANES)])
                    blv = pair(bot_v[b, i, pl.ds(q * LANES, LANES)])
                    brv = pair(bot_v[b, i, pl.ds(depth // 2 + q * LANES, LANES)])
                    out_v[b, i, pl.ds(c, LANES)] = (
                        w00 * tlv[0] + w01 * trv[0] +
                        w10 * blv[0] + w11 * brv[0])
                    out_v[b, i, pl.ds(c + LANES, LANES)] = (
                        w00 * tlv[1] + w01 * trv[1] +
                        w10 * blv[1] + w11 * brv[1])

            pltpu.async_copy(out_v.at[b], out_hbm.at[pl.ds(cb, CH)], osem)

        wait_store(0)
        wait_store(1)

    return sc_kernel


def _tc_relayout(flat_out, B, NB, C):
    BB = 8
    nblk = NB // BB

    def body(x_ref, o_ref):
        o_ref[...] = x_ref[...].reshape(o_ref.shape)

    return pl.pallas_call(
        body,
        grid=(B * nblk,),
        in_specs=[pl.BlockSpec((BB * POOL * POOL, C), lambda i: (i, 0))],
        out_specs=pl.BlockSpec((1, BB, POOL, POOL, C),
                               lambda i: (i // nblk, i % nblk, 0, 0, 0)),
        out_shape=jax.ShapeDtypeStruct((B, NB, POOL, POOL, C), jnp.float32),
    )(flat_out)


def kernel(inputs, proposals):
    B, H, W, C = inputs.shape
    NB = proposals.shape[1]
    n = B * NB

    boxes = proposals.reshape(-1, 4)
    y1, x1, y2, x2 = boxes[:, 0], boxes[:, 1], boxes[:, 2], boxes[:, 3]
    grid = jnp.arange(POOL, dtype=jnp.float32)
    h_scale = (y2 - y1) * (H - 1) / (POOL - 1)
    w_scale = (x2 - x1) * (W - 1) / (POOL - 1)
    in_y = y1[:, None] * (H - 1) + grid[None, :] * h_scale[:, None]
    in_x = x1[:, None] * (W - 1) + grid[None, :] * w_scale[:, None]
    valid_y = (in_y >= 0) & (in_y <= H - 1)
    valid_x = (in_x >= 0) & (in_x <= W - 1)
    top_y = jnp.floor(in_y)
    left_x = jnp.floor(in_x)
    yl = in_y - top_y
    xl = in_x - left_x
    ti = jnp.clip(top_y, 0, H - 1).astype(jnp.int32)
    bi = jnp.clip(jnp.ceil(in_y), 0, H - 1).astype(jnp.int32)
    li = jnp.clip(left_x, 0, W - 1).astype(jnp.int32)

    bflat = (jnp.arange(n, dtype=jnp.int32) // NB) * (H * W)
    idx_t = bflat[:, None, None] + ti[:, :, None] * W + li[:, None, :]
    idx_b = bflat[:, None, None] + bi[:, :, None] * W + li[:, None, :]

    v = (valid_y[:, :, None] & valid_x[:, None, :]).astype(jnp.float32)
    ylc = yl[:, :, None]
    xlc = xl[:, None, :]
    w00 = (1.0 - ylc) * (1.0 - xlc) * v
    w01 = (1.0 - ylc) * xlc * v
    w10 = ylc * (1.0 - xlc) * v
    w11 = ylc * xlc * v

    s = n * POOL * POOL
    s_pad = ((s + NW * CH - 1) // (NW * CH)) * (NW * CH)
    chunks_per_tile = s_pad // NW // CH
    per_tile = s_pad // NW

    n_id = (s - CH) // CH + 1
    rep = s_pad // CH - n_id

    def tiled(a):
        a = a.reshape(-1)
        a = jnp.concatenate([a[:n_id * CH], jnp.tile(a[s - CH:s], rep)])
        return a.reshape(NW, per_tile)

    p = jnp.arange(C)
    ch_perm = (p // 32) * 32 + (p % 2) * 16 + (p % 32) // 2
    flat = inputs.reshape(B * H * W, C)[:, ch_perm]
    table = jnp.concatenate(
        [flat, jnp.roll(flat, -1, axis=0)], axis=1).astype(jnp.bfloat16)
    table = jax.lax.bitcast_convert_type(
        table.reshape(B * H * W, C, 2), jnp.int32)

    sc_kernel = _build_sc_kernel(B * H * W, C, s, s_pad, chunks_per_tile)
    out = sc_kernel(table, tiled(idx_t), tiled(idx_b),
                    tiled(w00), tiled(w01), tiled(w10), tiled(w11))
    return _tc_relayout(out, B, NB, C)


# speedup vs baseline: 2.2960x; 1.2088x over previous
import dataclasses
import functools

import jax
import jax.numpy as jnp
from jax import lax
from jax.experimental import pallas as pl
from jax.experimental.pallas import tpu as pltpu
from jax.experimental.pallas import tpu_sc as plsc

POOL = 7
NC = 2
NS = 16
NW = NC * NS
LANES = 16
CH = 32


def _build_sc_kernel(n_rows, depth, s_out, s_pad, chunks_per_tile):
    per_tile = s_pad // NW
    d2 = 2 * depth
    last_cb = s_out - CH

    cp = pltpu.CompilerParams()
    if "needs_layout_passes" in pltpu.CompilerParams.__dataclass_fields__:
        cp = dataclasses.replace(cp, needs_layout_passes=False)

    @functools.partial(
        pl.kernel,
        mesh=plsc.VectorSubcoreMesh(core_axis_name="c", subcore_axis_name="s"),
        compiler_params=cp,
        out_type=jax.ShapeDtypeStruct((s_out, depth), jnp.float32),
        scratch_types=[
            pltpu.VMEM((per_tile,), jnp.int32),
            pltpu.VMEM((per_tile,), jnp.int32),
            pltpu.VMEM((per_tile,), jnp.float32),
            pltpu.VMEM((per_tile,), jnp.float32),
            pltpu.VMEM((per_tile,), jnp.float32),
            pltpu.VMEM((per_tile,), jnp.float32),
            pltpu.VMEM((2, CH, depth), jnp.int32),
            pltpu.VMEM((2, CH, depth), jnp.int32),
            pltpu.VMEM((2, CH, depth), jnp.float32),
            pltpu.SemaphoreType.DMA,
            pltpu.SemaphoreType.DMA,
        ],
    )
    def sc_kernel(table_hbm, idxt_hbm, idxb_hbm, w00_hbm, w01_hbm, w10_hbm,
                  w11_hbm, out_hbm, idxt_v, idxb_v, w00_v, w01_v, w10_v, w11_v,
                  top_v, bot_v, out_v, gsem, osem):
        wid = lax.axis_index("s") * NC + lax.axis_index("c")
        base = wid * per_tile
        pltpu.sync_copy(idxt_hbm.at[wid], idxt_v)
        pltpu.sync_copy(idxb_hbm.at[wid], idxb_v)
        pltpu.sync_copy(w00_hbm.at[wid], w00_v)
        pltpu.sync_copy(w01_hbm.at[wid], w01_v)
        pltpu.sync_copy(w10_hbm.at[wid], w10_v)
        pltpu.sync_copy(w11_hbm.at[wid], w11_v)

        def issue_gathers(g, b):
            sl = pl.ds(g * CH, CH)
            pltpu.async_copy(table_hbm.at[idxt_v.at[sl]], top_v.at[b], gsem)
            pltpu.async_copy(table_hbm.at[idxb_v.at[sl]], bot_v.at[b], gsem)

        def wait_gathers(b):
            dummy = table_hbm.at[pl.ds(0, CH)]
            pltpu.make_async_copy(dummy, top_v.at[b], gsem).wait()
            pltpu.make_async_copy(dummy, bot_v.at[b], gsem).wait()

        def wait_store(b):
            pltpu.make_async_copy(
                out_v.at[b], out_hbm.at[pl.ds(0, CH)], osem).wait()

        issue_gathers(0, 0)

        @pl.loop(0, chunks_per_tile)
        def _chunk(g):
            b = lax.rem(g, 2)
            cb = jnp.minimum(base + g * CH, last_cb)

            @pl.when(g + 1 < chunks_per_tile)
            def _():
                issue_gathers(g + 1, 1 - b)

            wait_gathers(b)

            @pl.when(g >= 2)
            def _():
                wait_store(b)

            @pl.loop(0, CH)
            def _sample(i):
                sv = jnp.full((LANES,), g * CH + i, jnp.int32)
                w00 = plsc.load_gather(w00_v, [sv])
                w01 = plsc.load_gather(w01_v, [sv])
                w10 = plsc.load_gather(w10_v, [sv])
                w11 = plsc.load_gather(w11_v, [sv])

                for q in range(depth // (2 * LANES)):
                    c = q * 2 * LANES

                    def pair(ref):
                        v = plsc.bitcast(ref, jnp.bfloat16)
                        return plsc.unpack(
                            v, format=plsc.PackFormat.INTERLEAVED)

                    tlv = pair(top_v[b, i, pl.ds(q * LANES, LANES)])
                    trv = pair(top_v[b, i, pl.ds(depth // 2 + q * LANES, LANES)])
                    blv = pair(bot_v[b, i, pl.ds(q * LANES, LANES)])
                    brv = pair(bot_v[b, i, pl.ds(depth // 2 + q * LANES, LANES)])
                    out_v[b, i, pl.ds(c, LANES)] = (
                        w00 * tlv[0] + w01 * trv[0] +
                        w10 * blv[0] + w11 * brv[0])
                    out_v[b, i, pl.ds(c + LANES, LANES)] = (
                        w00 * tlv[1] + w01 * trv[1] +
                        w10 * blv[1] + w11 * brv[1])

            pltpu.async_copy(out_v.at[b], out_hbm.at[pl.ds(cb, CH)], osem)

        wait_store(0)
        wait_store(1)

    return sc_kernel


def _tc_relayout(flat_out, B, NB, C):
    BB = 40
    nblk = NB // BB

    def body(x_ref, o_ref):
        o_ref[...] = x_ref[...].reshape(o_ref.shape)

    return pl.pallas_call(
        body,
        grid=(B * nblk,),
        in_specs=[pl.BlockSpec((BB * POOL * POOL, C), lambda i: (i, 0))],
        out_specs=pl.BlockSpec((1, BB, POOL, POOL, C),
                               lambda i: (i // nblk, i % nblk, 0, 0, 0)),
        out_shape=jax.ShapeDtypeStruct((B, NB, POOL, POOL, C), jnp.float32),
    )(flat_out)


def kernel(inputs, proposals):
    B, H, W, C = inputs.shape
    NB = proposals.shape[1]
    n = B * NB

    boxes = proposals.reshape(-1, 4)
    y1, x1, y2, x2 = boxes[:, 0], boxes[:, 1], boxes[:, 2], boxes[:, 3]
    grid = jnp.arange(POOL, dtype=jnp.float32)
    h_scale = (y2 - y1) * (H - 1) / (POOL - 1)
    w_scale = (x2 - x1) * (W - 1) / (POOL - 1)
    in_y = y1[:, None] * (H - 1) + grid[None, :] * h_scale[:, None]
    in_x = x1[:, None] * (W - 1) + grid[None, :] * w_scale[:, None]
    valid_y = (in_y >= 0) & (in_y <= H - 1)
    valid_x = (in_x >= 0) & (in_x <= W - 1)
    top_y = jnp.floor(in_y)
    left_x = jnp.floor(in_x)
    yl = in_y - top_y
    xl = in_x - left_x
    ti = jnp.clip(top_y, 0, H - 1).astype(jnp.int32)
    bi = jnp.clip(jnp.ceil(in_y), 0, H - 1).astype(jnp.int32)
    li = jnp.clip(left_x, 0, W - 1).astype(jnp.int32)

    bflat = (jnp.arange(n, dtype=jnp.int32) // NB) * (H * W)
    idx_t = bflat[:, None, None] + ti[:, :, None] * W + li[:, None, :]
    idx_b = bflat[:, None, None] + bi[:, :, None] * W + li[:, None, :]

    v = (valid_y[:, :, None] & valid_x[:, None, :]).astype(jnp.float32)
    ylc = yl[:, :, None]
    xlc = xl[:, None, :]
    w00 = (1.0 - ylc) * (1.0 - xlc) * v
    w01 = (1.0 - ylc) * xlc * v
    w10 = ylc * (1.0 - xlc) * v
    w11 = ylc * xlc * v

    s = n * POOL * POOL
    s_pad = ((s + NW * CH - 1) // (NW * CH)) * (NW * CH)
    chunks_per_tile = s_pad // NW // CH
    per_tile = s_pad // NW

    n_id = (s - CH) // CH + 1
    rep = s_pad // CH - n_id

    def tiled(a):
        a = a.reshape(-1)
        a = jnp.concatenate([a[:n_id * CH], jnp.tile(a[s - CH:s], rep)])
        return a.reshape(NW, per_tile)

    p = jnp.arange(C)
    ch_perm = (p // 32) * 32 + (p % 2) * 16 + (p % 32) // 2
    flat = inputs.reshape(B * H * W, C)[:, ch_perm]
    table = jnp.concatenate(
        [flat, jnp.roll(flat, -1, axis=0)], axis=1).astype(jnp.bfloat16)
    table = jax.lax.bitcast_convert_type(
        table.reshape(B * H * W, C, 2), jnp.int32)

    sc_kernel = _build_sc_kernel(B * H * W, C, s, s_pad, chunks_per_tile)
    out = sc_kernel(table, tiled(idx_t), tiled(idx_b),
                    tiled(w00), tiled(w01), tiled(w10), tiled(w11))
    return _tc_relayout(out, B, NB, C)
